# Initial kernel scaffold; baseline (speedup 1.0000x reference)
import jax
import jax.numpy as jnp
from jax.experimental import pallas as pl


def kernel(x, pos, batch, W1, b1, W2, b2):
    raise NotImplementedError("write your pallas kernel here")


# speedup vs baseline: 1.5411x; 1.5411x over previous
import functools

import jax
import jax.numpy as jnp
from jax.experimental import pallas as pl
from jax.experimental.pallas import tpu as pltpu

_RATIO = 0.125
_R2 = 0.2 * 0.2
_K = 64
_LANE = 128
_NEG_BIG = -jnp.inf


def _fps_kernel(nloop, nvalid, px_ref, py_ref, pz_ref, b_ref,
                idx_ref, sx_ref, sy_ref, sz_ref, sb_ref):
    shape = px_ref.shape
    nidx = (jax.lax.broadcasted_iota(jnp.int32, shape, 0) * _LANE
            + jax.lax.broadcasted_iota(jnp.int32, shape, 1))
    valid = nidx < nvalid
    d0 = jnp.where(valid, jnp.inf, -jnp.inf)

    oshape = idx_ref.shape
    oidx = (jax.lax.broadcasted_iota(jnp.int32, oshape, 0) * _LANE
            + jax.lax.broadcasted_iota(jnp.int32, oshape, 1))

    px = px_ref[...]
    py = py_ref[...]
    pz = pz_ref[...]
    bt = b_ref[...]

    def body(i, carry):
        d, idxs, sx, sy, sz, sb = carry
        m = jnp.max(d, axis=1, keepdims=True)
        m = jnp.max(m, axis=0, keepdims=True)
        cand = jnp.where(d == m, nidx, jnp.int32(2**30))
        nxt = jnp.min(cand, axis=1, keepdims=True)
        nxt = jnp.min(nxt, axis=0, keepdims=True)
        eq = nidx == nxt
        zf = jnp.zeros(shape, jnp.float32)
        cx = jnp.sum(jnp.where(eq, px, zf), axis=1, keepdims=True)
        cx = jnp.sum(cx, axis=0, keepdims=True)
        cy = jnp.sum(jnp.where(eq, py, zf), axis=1, keepdims=True)
        cy = jnp.sum(cy, axis=0, keepdims=True)
        cz = jnp.sum(jnp.where(eq, pz, zf), axis=1, keepdims=True)
        cz = jnp.sum(cz, axis=0, keepdims=True)
        cb = jnp.sum(jnp.where(eq, bt, jnp.zeros(shape, jnp.int32)),
                     axis=1, keepdims=True)
        cb = jnp.sum(cb, axis=0, keepdims=True)
        dx = px - cx
        dy = py - cy
        dz = pz - cz
        dist = (dx * dx + dy * dy) + dz * dz
        d = jnp.minimum(d, dist)
        here = oidx == i
        idxs = jnp.where(here, nxt, idxs)
        sx = jnp.where(here, cx, sx)
        sy = jnp.where(here, cy, sy)
        sz = jnp.where(here, cz, sz)
        sb = jnp.where(here, cb, sb)
        return d, idxs, sx, sy, sz, sb

    init = (d0,
            jnp.zeros(oshape, jnp.int32),
            jnp.zeros(oshape, jnp.float32),
            jnp.zeros(oshape, jnp.float32),
            jnp.zeros(oshape, jnp.float32),
            jnp.zeros(oshape, jnp.int32))
    _, idxs, sx, sy, sz, sb = jax.lax.fori_loop(0, nloop, body, init)
    idx_ref[...] = idxs
    sx_ref[...] = sx
    sy_ref[...] = sy
    sz_ref[...] = sz
    sb_ref[...] = sb


def _fps(px, py, pz, batch2d, nloop, nvalid, orows):
    rows = px.shape[0]
    out_shapes = (
        jax.ShapeDtypeStruct((orows, _LANE), jnp.int32),
        jax.ShapeDtypeStruct((orows, _LANE), jnp.float32),
        jax.ShapeDtypeStruct((orows, _LANE), jnp.float32),
        jax.ShapeDtypeStruct((orows, _LANE), jnp.float32),
        jax.ShapeDtypeStruct((orows, _LANE), jnp.int32),
    )
    return pl.pallas_call(
        functools.partial(_fps_kernel, nloop, nvalid),
        out_shape=out_shapes,
    )(px, py, pz, batch2d)


def _knn_kernel(nvalid, k, px_ref, py_ref, pz_ref, cx_ref, cy_ref, cz_ref,
                idx_ref, dsel_ref):
    rows = px_ref.shape[0]
    bc = cx_ref.shape[0]
    nidx = (jax.lax.broadcasted_iota(jnp.int32, (1, rows, _LANE), 1) * _LANE
            + jax.lax.broadcasted_iota(jnp.int32, (1, rows, _LANE), 2))
    px = px_ref[...].reshape(1, rows, _LANE)
    py = py_ref[...].reshape(1, rows, _LANE)
    pz = pz_ref[...].reshape(1, rows, _LANE)
    cx = cx_ref[...].reshape(bc, 1, 1)
    cy = cy_ref[...].reshape(bc, 1, 1)
    cz = cz_ref[...].reshape(bc, 1, 1)
    dx = px - cx
    dy = py - cy
    dz = pz - cz
    d = (dx * dx + dy * dy) + dz * dz
    d = jnp.where(nidx < nvalid, d, jnp.inf)

    kiota = jax.lax.broadcasted_iota(jnp.int32, (bc, k), 1)

    def body(j, carry):
        d, sel_idx, sel_d = carry
        m = jnp.min(d, axis=2, keepdims=True)
        m = jnp.min(m, axis=1, keepdims=True)
        cand = jnp.where(d == m, nidx, jnp.int32(2**30))
        nxt = jnp.min(cand, axis=2, keepdims=True)
        nxt = jnp.min(nxt, axis=1, keepdims=True)
        here = kiota == j
        sel_idx = jnp.where(here, nxt.reshape(bc, 1), sel_idx)
        sel_d = jnp.where(here, m.reshape(bc, 1), sel_d)
        d = jnp.where(nidx == nxt, jnp.inf, d)
        return d, sel_idx, sel_d

    _, sel_idx, sel_d = jax.lax.fori_loop(
        0, k, body,
        (d, jnp.zeros((bc, k), jnp.int32), jnp.zeros((bc, k), jnp.float32)))
    idx_ref[...] = sel_idx
    dsel_ref[...] = sel_d


def _knn(px, py, pz, cx, cy, cz, nvalid, k, bc):
    rows = px.shape[0]
    ns = cx.shape[0]
    grid = (ns // bc,)
    full = pl.BlockSpec((rows, _LANE), lambda i: (0, 0))
    cspec = pl.BlockSpec((bc, 1), lambda i: (i, 0))
    return pl.pallas_call(
        functools.partial(_knn_kernel, nvalid, k),
        grid=grid,
        in_specs=[full, full, full, cspec, cspec, cspec],
        out_specs=(pl.BlockSpec((bc, k), lambda i: (i, 0)),
                   pl.BlockSpec((bc, k), lambda i: (i, 0))),
        out_shape=(jax.ShapeDtypeStruct((ns, k), jnp.int32),
                   jax.ShapeDtypeStruct((ns, k), jnp.float32)),
        compiler_params=pltpu.CompilerParams(
            dimension_semantics=("arbitrary",)),
    )(px, py, pz, cx, cy, cz)


def _dense_kernel(x_ref, posp_ref, w1x_ref, w1p_ref, b1_ref, g_ref):
    g = jnp.dot(x_ref[...], w1x_ref[...], preferred_element_type=jnp.float32)
    g = g + jnp.dot(posp_ref[...], w1p_ref[...],
                    preferred_element_type=jnp.float32)
    g_ref[...] = g + b1_ref[...]


def _dense(xp, pospad, w1x, w1p_pad, b1, blk):
    n, c = xp.shape
    h = w1x.shape[1]
    grid = (n // blk,)
    return pl.pallas_call(
        _dense_kernel,
        grid=grid,
        in_specs=[pl.BlockSpec((blk, c), lambda i: (i, 0)),
                  pl.BlockSpec((blk, _LANE), lambda i: (i, 0)),
                  pl.BlockSpec((c, h), lambda i: (0, 0)),
                  pl.BlockSpec((_LANE, h), lambda i: (0, 0)),
                  pl.BlockSpec((1, h), lambda i: (0, 0))],
        out_specs=pl.BlockSpec((blk, h), lambda i: (i, 0)),
        out_shape=jax.ShapeDtypeStruct((n, h), jnp.float32),
        compiler_params=pltpu.CompilerParams(
            dimension_semantics=("arbitrary",)),
    )(xp, pospad, w1x, w1p_pad, b1)


def _conv_kernel(k, bc, g_ref, idx_ref, dsel_ref, cx_ref, cy_ref, cz_ref,
                 w1p_ref, w2_ref, b2_ref, out_ref, h_scr, m_scr):
    h = g_ref.shape[1]
    w0 = w1p_ref[0:1, :]
    w1 = w1p_ref[1:2, :]
    w2r = w1p_ref[2:3, :]
    zrow = jnp.zeros((1, h), jnp.float32)
    for i in range(bc):
        cxi = cx_ref[i, 0]
        cyi = cy_ref[i, 0]
        czi = cz_ref[i, 0]
        cw = cxi * w0 + cyi * w1 + czi * w2r

        def body(j, _):
            src = idx_ref[i, j]
            h_scr[pl.ds(j, 1), :] = g_ref[pl.ds(src, 1), :]
            dv = dsel_ref[i, j]
            m_scr[pl.ds(j, 1), :] = zrow + jnp.where(
                dv <= _R2, 0.0, -jnp.inf)
            return 0

        jax.lax.fori_loop(0, k, body, 0)
        h1 = jnp.maximum(h_scr[...] - cw, 0.0)
        z = jnp.dot(h1, w2_ref[...], preferred_element_type=jnp.float32)
        z = jnp.maximum(z + b2_ref[...], 0.0)
        z = z + m_scr[...]
        out_ref[pl.ds(i, 1), :] = jnp.max(z, axis=0, keepdims=True)


def _conv(g, nbr_idx, dsel, cx, cy, cz, w1p, w2, b2, k, bc):
    n, h = g.shape
    ns = nbr_idx.shape[0]
    grid = (ns // bc,)
    smem = pltpu.SMEM
    return pl.pallas_call(
        functools.partial(_conv_kernel, k, bc),
        grid=grid,
        in_specs=[pl.BlockSpec((n, h), lambda i: (0, 0)),
                  pl.BlockSpec((bc, k), lambda i: (i, 0), memory_space=smem),
                  pl.BlockSpec((bc, k), lambda i: (i, 0), memory_space=smem),
                  pl.BlockSpec((bc, 1), lambda i: (i, 0), memory_space=smem),
                  pl.BlockSpec((bc, 1), lambda i: (i, 0), memory_space=smem),
                  pl.BlockSpec((bc, 1), lambda i: (i, 0), memory_space=smem),
                  pl.BlockSpec((3, h), lambda i: (0, 0)),
                  pl.BlockSpec((h, h), lambda i: (0, 0)),
                  pl.BlockSpec((1, h), lambda i: (0, 0))],
        out_specs=pl.BlockSpec((bc, h), lambda i: (i, 0)),
        out_shape=jax.ShapeDtypeStruct((ns, h), jnp.float32),
        scratch_shapes=[pltpu.VMEM((k, h), jnp.float32),
                        pltpu.VMEM((k, h), jnp.float32)],
        compiler_params=pltpu.CompilerParams(
            dimension_semantics=("arbitrary",)),
    )(g, nbr_idx, dsel, cx, cy, cz, w1p, w2, b2)


def kernel(x, pos, batch, W1, b1, W2, b2):
    n, c = x.shape
    hdim = W1.shape[1]
    s = int(n * _RATIO)
    rows = (n + _LANE - 1) // _LANE
    rows = ((rows + 7) // 8) * 8
    npad = rows * _LANE
    orows = ((s + _LANE - 1) // _LANE + 7) // 8 * 8

    def plane(v):
        return jnp.pad(v, (0, npad - n)).reshape(rows, _LANE)

    px = plane(pos[:, 0])
    py = plane(pos[:, 1])
    pz = plane(pos[:, 2])
    b2d = jnp.pad(batch, (0, npad - n)).reshape(rows, _LANE)

    orows = ((s + _LANE - 1) // _LANE)
    orows = ((orows + 7) // 8) * 8
    idx2d, sx, sy, sz, sb = _fps(px, py, pz, b2d, s, n, orows)
    idx_flat = idx2d.reshape(-1)[:s]
    cxf = sx.reshape(-1)[:s]
    cyf = sy.reshape(-1)[:s]
    czf = sz.reshape(-1)[:s]
    batch_new = sb.reshape(-1)[:s]
    pos_new = jnp.stack([cxf, cyf, czf], axis=1)

    bc = 8
    spad = ((s + bc - 1) // bc) * bc
    far = jnp.float32(1e9)
    cxp = jnp.concatenate([cxf, jnp.full((spad - s,), far)]).reshape(spad, 1)
    cyp = jnp.concatenate([cyf, jnp.full((spad - s,), far)]).reshape(spad, 1)
    czp = jnp.concatenate([czf, jnp.full((spad - s,), far)]).reshape(spad, 1)
    nbr_idx, dsel = _knn(px, py, pz, cxp, cyp, czp, n, _K, bc)

    xp = jnp.pad(x, ((0, npad - n), (0, 0)))
    pospad = jnp.pad(pos, ((0, npad - n), (0, _LANE - 3)))
    w1x = W1[:c]
    w1p = W1[c:]
    w1p_pad = jnp.pad(w1p, ((0, _LANE - 3), (0, 0)))
    g = _dense(xp, pospad, w1x, w1p_pad, b1.reshape(1, hdim), 512)

    out = _conv(g, nbr_idx, dsel, cxp, cyp, czp, w1p, W2,
                b2.reshape(1, hdim), _K, bc)
    x_dest = out[:s]
    return x_dest, pos_new, batch_new
